# baseline (speedup 1.0000x reference)
import jax
import jax.numpy as jnp
from jax.experimental import pallas as pl
from jax.experimental.pallas import tpu as pltpu

_A = 8400
_AP = 8448
_NP = 64
_NC = 80
_TOPK = 9
_EPS = 1e-9
_L0 = 6400
_L1 = 1600
_INF = 3.0e38
_BIGI = 1 << 30


def _top9(d, width):
    cidx = jax.lax.broadcasted_iota(jnp.int32, (_NP, width), 1)
    s = jnp.zeros((_NP, width), jnp.float32)
    for _ in range(_TOPK):
        m = jnp.min(d, axis=1, keepdims=True)
        j = jnp.min(jnp.where(d == m, cidx, _BIGI), axis=1, keepdims=True)
        pick = cidx == j
        s = s + pick.astype(jnp.float32)
        d = jnp.where(pick, _INF, d)
    return s


def _atss_body(anc_ref, gt_ref, gtr_ref, lab_ref, box_ref, sco_ref):
    a = anc_ref[:, :]
    ax0 = a[0:1, :]
    ay0 = a[1:2, :]
    ax1 = a[2:3, :]
    ay1 = a[3:4, :]
    g = gt_ref[0]
    gx0 = g[:, 0:1]
    gy0 = g[:, 1:2]
    gx1 = g[:, 2:3]
    gy1 = g[:, 3:4]
    glab = g[:, 4:5]
    gmask = g[:, 5:6]

    acx = (ax0 + ax1) * 0.5
    acy = (ay0 + ay1) * 0.5
    aarea = (ax1 - ax0) * (ay1 - ay0)
    gcx = (gx0 + gx1) * 0.5
    gcy = (gy0 + gy1) * 0.5
    garea = (gx1 - gx0) * (gy1 - gy0)

    dx = gcx - acx
    dy = gcy - acy
    aidx = jax.lax.broadcasted_iota(jnp.int32, (1, _AP), 1)
    dist = jnp.where(aidx < _A, jnp.sqrt(dx * dx + dy * dy), _INF)

    sel0 = _top9(dist[:, 0:_L0], _L0)
    dhi = dist[:, _L0:_AP]
    cidx = jax.lax.broadcasted_iota(jnp.int32, (_NP, _AP - _L0), 1)
    sel1 = _top9(jnp.where(cidx < _L1, dhi, _INF), _AP - _L0)
    sel2 = _top9(jnp.where(cidx >= _L1, dhi, _INF), _AP - _L0)
    sel = jnp.concatenate([sel0, sel1 + sel2], axis=1)

    inter = (jnp.maximum(jnp.minimum(gx1, ax1) - jnp.maximum(gx0, ax0), 0.0)
             * jnp.maximum(jnp.minimum(gy1, ay1) - jnp.maximum(gy0, ay0), 0.0))
    iou = inter / (garea + aarea - inter + _EPS)

    selm = sel * gmask
    iou_c = iou * selm
    mean = jnp.sum(iou_c, axis=1, keepdims=True) * (1.0 / (3 * _TOPK))
    dvar = iou_c - mean
    var = jnp.sum(sel * dvar * dvar, axis=1, keepdims=True) * (1.0 / (3 * _TOPK - 1))
    thr = mean + jnp.sqrt(jnp.maximum(var, 0.0))
    topk_f = jnp.where(iou_c > thr, selm, jnp.zeros_like(selm))

    m_in = jnp.minimum(jnp.minimum(acx - gx0, acy - gy0),
                       jnp.minimum(gx1 - acx, gy1 - acy))
    maskp = topk_f * (m_in > _EPS).astype(jnp.float32) * gmask

    colsum = jnp.sum(maskp, axis=0, keepdims=True)
    multi = colsum > 1.0
    gidx = jax.lax.broadcasted_iota(jnp.int32, (_NP, _AP), 0)
    miou = jnp.max(iou, axis=0, keepdims=True)
    firstmax = jnp.min(jnp.where(iou == miou, gidx, _BIGI), axis=0,
                       keepdims=True)
    ismax = (gidx == firstmax).astype(jnp.float32)
    maskp2 = jnp.where(multi, ismax, maskp)

    possum = jnp.sum(maskp2, axis=0, keepdims=True)
    pos = possum > 0.0
    firstpos = jnp.min(jnp.where(maskp2 > 0.0, gidx, _BIGI), axis=0,
                       keepdims=True)
    assigned = jnp.where(pos, firstpos, jnp.zeros_like(firstpos))

    onehot = (gidx == assigned).astype(jnp.float32)
    labi = jnp.sum(onehot * glab, axis=0, keepdims=True).astype(jnp.int32)
    labi = jnp.where(pos, labi, jnp.full_like(labi, -1))
    lab_ref[0] = labi

    lab_c = jnp.transpose(labi, (1, 0))
    asg_c = jnp.transpose(assigned, (1, 0))
    gr = gtr_ref[0]
    gidx_r = jax.lax.broadcasted_iota(jnp.int32, (_AP, _NP), 1)
    onehot_c = (gidx_r == asg_c).astype(jnp.float32)
    for j in range(4):
        box_ref[0, :, j:j + 1] = jnp.sum(onehot_c * gr[j:j + 1, :], axis=1,
                                         keepdims=True)
    cls = jax.lax.broadcasted_iota(jnp.int32, (_AP, _NC), 1)
    sco_ref[0] = jnp.where(lab_c == cls, jnp.float32(1.0), jnp.float32(0.0))


def kernel(anchor_bboxes, num_anchors_list, gt_labels, gt_bboxes, pad_gt_mask,
           bg_index):
    B, n, _ = gt_bboxes.shape
    anc = jnp.zeros((8, _AP), jnp.float32).at[:4, :_A].set(
        anchor_bboxes.astype(jnp.float32).T)
    packed = jnp.concatenate(
        [gt_bboxes.astype(jnp.float32),
         gt_labels.astype(jnp.float32),
         pad_gt_mask.astype(jnp.float32),
         jnp.zeros((B, n, 2), jnp.float32)], axis=2)
    packed = jnp.pad(packed, ((0, 0), (0, _NP - n), (0, 0)))
    packed_r = jnp.transpose(packed, (0, 2, 1))

    cparams = pltpu.CompilerParams(dimension_semantics=("parallel",))
    lab, box, sco = pl.pallas_call(
        _atss_body,
        grid=(B,),
        in_specs=[
            pl.BlockSpec((8, _AP), lambda b: (0, 0)),
            pl.BlockSpec((1, _NP, 8), lambda b: (b, 0, 0)),
            pl.BlockSpec((1, 8, _NP), lambda b: (b, 0, 0)),
        ],
        out_specs=[
            pl.BlockSpec((1, 1, _AP), lambda b: (b, 0, 0)),
            pl.BlockSpec((1, _AP, 4), lambda b: (b, 0, 0)),
            pl.BlockSpec((1, _AP, _NC), lambda b: (b, 0, 0)),
        ],
        out_shape=[
            jax.ShapeDtypeStruct((B, 1, _AP), jnp.int32),
            jax.ShapeDtypeStruct((B, _AP, 4), jnp.float32),
            jax.ShapeDtypeStruct((B, _AP, _NC), jnp.float32),
        ],
        compiler_params=cparams,
    )(anc, packed, packed_r)

    labels = lab[:, 0, :_A]
    labels = jnp.where(labels < 0, bg_index, labels).astype(jnp.int32)
    return labels, box[:, :_A, :], sco[:, :_A, :]


# speedup vs baseline: 1.2012x; 1.2012x over previous
import jax
import jax.numpy as jnp
from jax.experimental import pallas as pl
from jax.experimental.pallas import tpu as pltpu

_A = 8400
_AP = 8448
_NP = 64
_NC = 80
_TOPK = 9
_EPS = 1e-9
_L0 = 6400
_L1 = 1600
_INF = 3.0e38
_BIGI = 1 << 30


def _top9(d, width):
    cidx = jax.lax.broadcasted_iota(jnp.int32, (_NP, width), 1)
    s = jnp.zeros((_NP, width), jnp.float32)
    for _ in range(_TOPK):
        m = jnp.min(d, axis=1, keepdims=True)
        j = jnp.min(jnp.where(d == m, cidx, _BIGI), axis=1, keepdims=True)
        pick = cidx == j
        s = s + pick.astype(jnp.float32)
        d = jnp.where(pick, _INF, d)
    return s


def _atss_body(anc_ref, gt_ref, gtr_ref, lab_ref, box_ref, sco_ref):
    a = anc_ref[:, :]
    ax0 = a[0:1, :]
    ay0 = a[1:2, :]
    ax1 = a[2:3, :]
    ay1 = a[3:4, :]
    g = gt_ref[0]
    gx0 = g[:, 0:1]
    gy0 = g[:, 1:2]
    gx1 = g[:, 2:3]
    gy1 = g[:, 3:4]
    glab = g[:, 4:5]
    gmask = g[:, 5:6]

    acx = (ax0 + ax1) * 0.5
    acy = (ay0 + ay1) * 0.5
    aarea = (ax1 - ax0) * (ay1 - ay0)
    gcx = (gx0 + gx1) * 0.5
    gcy = (gy0 + gy1) * 0.5
    garea = (gx1 - gx0) * (gy1 - gy0)

    dx = gcx - acx
    dy = gcy - acy
    aidx = jax.lax.broadcasted_iota(jnp.int32, (1, _AP), 1)
    dist = jnp.where(aidx < _A, jnp.sqrt(dx * dx + dy * dy), _INF)

    sel0 = _top9(dist[:, 0:_L0], _L0)
    dhi = dist[:, _L0:_AP]
    cidx = jax.lax.broadcasted_iota(jnp.int32, (_NP, _AP - _L0), 1)
    sel1 = _top9(jnp.where(cidx < _L1, dhi, _INF), _AP - _L0)
    sel2 = _top9(jnp.where(cidx >= _L1, dhi, _INF), _AP - _L0)
    sel = jnp.concatenate([sel0, sel1 + sel2], axis=1)

    inter = (jnp.maximum(jnp.minimum(gx1, ax1) - jnp.maximum(gx0, ax0), 0.0)
             * jnp.maximum(jnp.minimum(gy1, ay1) - jnp.maximum(gy0, ay0), 0.0))
    iou = inter / (garea + aarea - inter + _EPS)

    selm = sel * gmask
    iou_c = iou * selm
    mean = jnp.sum(iou_c, axis=1, keepdims=True) * (1.0 / (3 * _TOPK))
    dvar = iou_c - mean
    var = jnp.sum(sel * dvar * dvar, axis=1, keepdims=True) * (1.0 / (3 * _TOPK - 1))
    thr = mean + jnp.sqrt(jnp.maximum(var, 0.0))
    topk_f = jnp.where(iou_c > thr, selm, jnp.zeros_like(selm))

    m_in = jnp.minimum(jnp.minimum(acx - gx0, acy - gy0),
                       jnp.minimum(gx1 - acx, gy1 - acy))
    maskp = topk_f * (m_in > _EPS).astype(jnp.float32) * gmask

    colsum = jnp.sum(maskp, axis=0, keepdims=True)
    multi = colsum > 1.0
    gidx = jax.lax.broadcasted_iota(jnp.int32, (_NP, _AP), 0)
    miou = jnp.max(iou, axis=0, keepdims=True)
    firstmax = jnp.min(jnp.where(iou == miou, gidx, _BIGI), axis=0,
                       keepdims=True)
    ismax = (gidx == firstmax).astype(jnp.float32)
    maskp2 = jnp.where(multi, ismax, maskp)

    possum = jnp.sum(maskp2, axis=0, keepdims=True)
    pos = possum > 0.0
    firstpos = jnp.min(jnp.where(maskp2 > 0.0, gidx, _BIGI), axis=0,
                       keepdims=True)
    assigned = jnp.where(pos, firstpos, jnp.zeros_like(firstpos))

    onehot = (gidx == assigned).astype(jnp.float32)
    labi = jnp.sum(onehot * glab, axis=0, keepdims=True).astype(jnp.int32)
    labi = jnp.where(pos, labi, jnp.full_like(labi, -1))
    lab_ref[0] = labi

    combo = assigned * 128 + (labi + 1)
    combo_c = jnp.transpose(combo, (1, 0))[0:_A, :]
    asg_c = jax.lax.shift_right_logical(combo_c, 7)
    lab_c = jax.lax.rem(combo_c, 128) - 1
    gr = gtr_ref[0]
    gidx_r = jax.lax.broadcasted_iota(jnp.int32, (_A, _NP), 1)
    onehot_c = (gidx_r == asg_c).astype(jnp.float32)
    for j in range(4):
        box_ref[0, :, j:j + 1] = jnp.sum(onehot_c * gr[j:j + 1, :], axis=1,
                                         keepdims=True)
    cls = jax.lax.broadcasted_iota(jnp.int32, (_A, _NC), 1)
    sco_ref[0] = jnp.where(lab_c == cls, jnp.float32(1.0), jnp.float32(0.0))


def kernel(anchor_bboxes, num_anchors_list, gt_labels, gt_bboxes, pad_gt_mask,
           bg_index):
    B, n, _ = gt_bboxes.shape
    anc = jnp.zeros((8, _AP), jnp.float32).at[:4, :_A].set(
        anchor_bboxes.astype(jnp.float32).T)
    packed = jnp.concatenate(
        [gt_bboxes.astype(jnp.float32),
         gt_labels.astype(jnp.float32),
         pad_gt_mask.astype(jnp.float32),
         jnp.zeros((B, n, 2), jnp.float32)], axis=2)
    packed = jnp.pad(packed, ((0, 0), (0, _NP - n), (0, 0)))
    packed_r = jnp.transpose(packed, (0, 2, 1))

    cparams = pltpu.CompilerParams(dimension_semantics=("parallel",))
    lab, box, sco = pl.pallas_call(
        _atss_body,
        grid=(B,),
        in_specs=[
            pl.BlockSpec((8, _AP), lambda b: (0, 0)),
            pl.BlockSpec((1, _NP, 8), lambda b: (b, 0, 0)),
            pl.BlockSpec((1, 8, _NP), lambda b: (b, 0, 0)),
        ],
        out_specs=[
            pl.BlockSpec((1, 1, _AP), lambda b: (b, 0, 0)),
            pl.BlockSpec((1, _A, 4), lambda b: (b, 0, 0)),
            pl.BlockSpec((1, _A, _NC), lambda b: (b, 0, 0)),
        ],
        out_shape=[
            jax.ShapeDtypeStruct((B, 1, _AP), jnp.int32),
            jax.ShapeDtypeStruct((B, _A, 4), jnp.float32),
            jax.ShapeDtypeStruct((B, _A, _NC), jnp.float32),
        ],
        compiler_params=cparams,
    )(anc, packed, packed_r)

    labels = lab[:, 0, :_A]
    labels = jnp.where(labels < 0, bg_index, labels).astype(jnp.int32)
    return labels, box, sco


# speedup vs baseline: 1.4064x; 1.1708x over previous
import jax
import jax.numpy as jnp
from jax.experimental import pallas as pl
from jax.experimental.pallas import tpu as pltpu

_A = 8400
_AP = 8448
_NP = 64
_NC = 80
_TOPK = 9
_EPS = 1e-9
_L0 = 6400
_L1 = 1600
_INF = 3.0e38
_BIGI = 1 << 30


def _top9(d, width):
    cidx = jax.lax.broadcasted_iota(jnp.int32, (_NP, width), 1)
    s = jnp.zeros((_NP, width), jnp.float32)
    for _ in range(_TOPK):
        m = jnp.min(d, axis=1, keepdims=True)
        j = jnp.min(jnp.where(d == m, cidx, _BIGI), axis=1, keepdims=True)
        pick = cidx == j
        s = s + pick.astype(jnp.float32)
        d = jnp.where(pick, _INF, d)
    return s


def _atss_body(anc_ref, gt_ref, lab_ref, box_ref, sco_ref):
    a = anc_ref[:, :]
    ax0 = a[0:1, :]
    ay0 = a[1:2, :]
    ax1 = a[2:3, :]
    ay1 = a[3:4, :]
    g = gt_ref[0]
    gx0 = g[:, 0:1]
    gy0 = g[:, 1:2]
    gx1 = g[:, 2:3]
    gy1 = g[:, 3:4]
    glab = g[:, 4:5]
    gmask = g[:, 5:6]

    acx = (ax0 + ax1) * 0.5
    acy = (ay0 + ay1) * 0.5
    aarea = (ax1 - ax0) * (ay1 - ay0)
    gcx = (gx0 + gx1) * 0.5
    gcy = (gy0 + gy1) * 0.5
    garea = (gx1 - gx0) * (gy1 - gy0)

    dx = gcx - acx
    dy = gcy - acy
    aidx = jax.lax.broadcasted_iota(jnp.int32, (1, _AP), 1)
    dist = jnp.where(aidx < _A, jnp.sqrt(dx * dx + dy * dy), _INF)

    sel0 = _top9(dist[:, 0:_L0], _L0)
    dhi = dist[:, _L0:_AP]
    cidx = jax.lax.broadcasted_iota(jnp.int32, (_NP, _AP - _L0), 1)
    sel1 = _top9(jnp.where(cidx < _L1, dhi, _INF), _AP - _L0)
    sel2 = _top9(jnp.where(cidx >= _L1, dhi, _INF), _AP - _L0)
    sel = jnp.concatenate([sel0, sel1 + sel2], axis=1)

    inter = (jnp.maximum(jnp.minimum(gx1, ax1) - jnp.maximum(gx0, ax0), 0.0)
             * jnp.maximum(jnp.minimum(gy1, ay1) - jnp.maximum(gy0, ay0), 0.0))
    iou = inter / (garea + aarea - inter + _EPS)

    selm = sel * gmask
    iou_c = iou * selm
    mean = jnp.sum(iou_c, axis=1, keepdims=True) * (1.0 / (3 * _TOPK))
    dvar = iou_c - mean
    var = jnp.sum(sel * dvar * dvar, axis=1, keepdims=True) * (1.0 / (3 * _TOPK - 1))
    thr = mean + jnp.sqrt(jnp.maximum(var, 0.0))
    topk_f = jnp.where(iou_c > thr, selm, jnp.zeros_like(selm))

    m_in = jnp.minimum(jnp.minimum(acx - gx0, acy - gy0),
                       jnp.minimum(gx1 - acx, gy1 - acy))
    maskp = topk_f * (m_in > _EPS).astype(jnp.float32) * gmask

    colsum = jnp.sum(maskp, axis=0, keepdims=True)
    multi = colsum > 1.0
    gidx = jax.lax.broadcasted_iota(jnp.int32, (_NP, _AP), 0)
    miou = jnp.max(iou, axis=0, keepdims=True)
    firstmax = jnp.min(jnp.where(iou == miou, gidx, _BIGI), axis=0,
                       keepdims=True)
    ismax = (gidx == firstmax).astype(jnp.float32)
    maskp2 = jnp.where(multi, ismax, maskp)

    possum = jnp.sum(maskp2, axis=0, keepdims=True)
    pos = possum > 0.0
    firstpos = jnp.min(jnp.where(maskp2 > 0.0, gidx, _BIGI), axis=0,
                       keepdims=True)
    assigned = jnp.where(pos, firstpos, jnp.zeros_like(firstpos))

    onehot = (gidx == assigned).astype(jnp.float32)
    labi = jnp.sum(onehot * glab, axis=0, keepdims=True).astype(jnp.int32)
    labi = jnp.where(pos, labi, jnp.full_like(labi, -1))
    lab_ref[0] = labi

    rows = [jax.lax.bitcast_convert_type(labi, jnp.float32)]
    for j in range(4):
        rows.append(jnp.sum(onehot * g[:, j:j + 1], axis=0, keepdims=True))
    rows.append(jnp.zeros((3, _AP), jnp.float32))
    tr = jnp.transpose(jnp.concatenate(rows, axis=0), (1, 0))
    box_ref[0] = tr[0:_A, 1:5]
    lab_c = jax.lax.bitcast_convert_type(tr[0:_A, 0:1], jnp.int32)
    cls = jax.lax.broadcasted_iota(jnp.int32, (_A, _NC), 1)
    sco_ref[0] = jnp.where(lab_c == cls, jnp.float32(1.0), jnp.float32(0.0))


def kernel(anchor_bboxes, num_anchors_list, gt_labels, gt_bboxes, pad_gt_mask,
           bg_index):
    B, n, _ = gt_bboxes.shape
    anc = jnp.zeros((8, _AP), jnp.float32).at[:4, :_A].set(
        anchor_bboxes.astype(jnp.float32).T)
    packed = jnp.concatenate(
        [gt_bboxes.astype(jnp.float32),
         gt_labels.astype(jnp.float32),
         pad_gt_mask.astype(jnp.float32),
         jnp.zeros((B, n, 2), jnp.float32)], axis=2)
    packed = jnp.pad(packed, ((0, 0), (0, _NP - n), (0, 0)))

    cparams = pltpu.CompilerParams(dimension_semantics=("parallel",))
    lab, box, sco = pl.pallas_call(
        _atss_body,
        grid=(B,),
        in_specs=[
            pl.BlockSpec((8, _AP), lambda b: (0, 0)),
            pl.BlockSpec((1, _NP, 8), lambda b: (b, 0, 0)),
        ],
        out_specs=[
            pl.BlockSpec((1, 1, _AP), lambda b: (b, 0, 0)),
            pl.BlockSpec((1, _A, 4), lambda b: (b, 0, 0)),
            pl.BlockSpec((1, _A, _NC), lambda b: (b, 0, 0)),
        ],
        out_shape=[
            jax.ShapeDtypeStruct((B, 1, _AP), jnp.int32),
            jax.ShapeDtypeStruct((B, _A, 4), jnp.float32),
            jax.ShapeDtypeStruct((B, _A, _NC), jnp.float32),
        ],
        compiler_params=cparams,
    )(anc, packed)

    labels = lab[:, 0, :_A]
    labels = jnp.where(labels < 0, bg_index, labels).astype(jnp.int32)
    return labels, box, sco
